# baseline (speedup 1.0000x reference)
import functools

import jax
import jax.numpy as jnp
from jax import lax
from jax.experimental import pallas as pl
from jax.experimental.pallas import tpu as pltpu
from jax.experimental.pallas import tpu_sc as plsc


_BN = 256
_BK = 512
_BR = 256
_LANES = 128


_K = 8192
_NKS = _K // _BK


def _argmin_body(f_ref, ct_ref, out_ref, ctm2_s, csq_s):
    n = pl.program_id(0)

    @pl.when(n == 0)
    def _():
        ct = ct_ref[...]
        csq = jnp.sum(ct * ct, axis=0, keepdims=True)
        ctm2_s[...] = ct * (-2.0)
        csq_s[...] = jnp.broadcast_to(csq, (8, _K))

    f = f_ref[...]
    f_sq = jnp.sum(f * f, axis=1, keepdims=True)

    for r in range(_BN // _BR):
        rs = slice(r * _BR, (r + 1) * _BR)
        fr = f[rs, :]
        fsq_r = f_sq[rs, :]
        rmin = jnp.full((_BR, _LANES), jnp.inf, jnp.float32)
        rarg = jnp.zeros((_BR, _LANES), jnp.int32)
        for kb in range(_NKS):
            ksl = slice(kb * _BK, (kb + 1) * _BK)
            t = jnp.dot(fr, ctm2_s[:, ksl],
                        preferred_element_type=jnp.float32)
            csq_k = csq_s[0:1, ksl]
            for j in range(_BK // _LANES):
                sl = slice(j * _LANES, (j + 1) * _LANES)
                d2 = (fsq_r + csq_k[:, sl]) + t[:, sl]
                chunk = kb * (_BK // _LANES) + j
                lt = d2 < rmin
                rmin = jnp.minimum(d2, rmin)
                rarg = jnp.where(lt, chunk, rarg)
        gm = jnp.min(rmin, axis=1, keepdims=True)
        lane = lax.broadcasted_iota(jnp.int32, (_BR, _LANES), 1)
        fullidx = rarg * _LANES + lane
        cand = jnp.where(rmin == gm, fullidx, jnp.int32(2**30))
        out_ref[rs, :] = jnp.min(cand, axis=1, keepdims=True)


def _nearest_indices(features, ct):
    n, d = features.shape
    grid = (n // _BN,)
    return pl.pallas_call(
        _argmin_body,
        grid=grid,
        in_specs=[
            pl.BlockSpec((_BN, d), lambda i: (i, 0)),
            pl.BlockSpec((d, _K), lambda i: (0, 0)),
        ],
        out_specs=pl.BlockSpec((_BN, 1), lambda i: (i, 0)),
        out_shape=jax.ShapeDtypeStruct((n, 1), jnp.int32),
        scratch_shapes=[
            pltpu.VMEM((d, _K), jnp.float32),
            pltpu.VMEM((8, _K), jnp.float32),
        ],
        compiler_params=pltpu.CompilerParams(
            dimension_semantics=("arbitrary",),
        ),
    )(features, ct)


_NC = 2
_NS = 16
_NW = _NC * _NS
_CHUNK = 128


def _make_sc_gather(n, d):
    rows_per_w = n // (_NW * _CHUNK)
    b_per_w = rows_per_w * _CHUNK
    mesh = plsc.VectorSubcoreMesh(core_axis_name="c", subcore_axis_name="s")

    @functools.partial(
        pl.kernel,
        mesh=mesh,
        out_type=jax.ShapeDtypeStruct((n, d), jnp.float32),
        scratch_types=[
            pltpu.VMEM((rows_per_w, _CHUNK), jnp.int32),
            pltpu.VMEM((b_per_w, d), jnp.float32),
            pltpu.SemaphoreType.DMA,
        ],
        compiler_params=pltpu.CompilerParams(use_tc_tiling_on_sc=False),
    )
    def gather(codes_hbm, idx_hbm, out_hbm, idx_v, rows_v, sem):
        wid = lax.axis_index("s") * _NC + lax.axis_index("c")
        pltpu.sync_copy(idx_hbm.at[wid], idx_v)
        copies = [
            pltpu.async_copy(
                codes_hbm.at[idx_v.at[j]],
                rows_v.at[pl.ds(j * _CHUNK, _CHUNK)],
                sem,
            )
            for j in range(rows_per_w)
        ]
        for c in copies:
            c.wait()
        pltpu.sync_copy(rows_v, out_hbm.at[pl.ds(wid * b_per_w, b_per_w)])

    return gather


def kernel(features, codes):
    n, d = features.shape
    ct = codes.T
    h = n // 3
    gather = _make_sc_gather(h, d)
    outs = []
    for i in range(3):
        fh = lax.slice_in_dim(features, i * h, (i + 1) * h, axis=0)
        idx = _nearest_indices(fh, ct)
        idx2 = idx.reshape(_NW, h // (_NW * _CHUNK), _CHUNK)
        outs.append(gather(codes, idx2))
    return jnp.concatenate(outs, axis=0)


# speedup vs baseline: 1.1287x; 1.1287x over previous
import functools

import jax
import jax.numpy as jnp
from jax import lax
from jax.experimental import pallas as pl
from jax.experimental.pallas import tpu as pltpu
from jax.experimental.pallas import tpu_sc as plsc


_BN = 512
_BK = 512
_BR = 256
_LANES = 128


_K = 8192
_NKS = _K // _BK


def _argmin_body(f_ref, ct_ref, out_ref, ctm2_s, csq_s):
    n = pl.program_id(0)

    @pl.when(n == 0)
    def _():
        ct = ct_ref[...]
        csq = jnp.sum(ct * ct, axis=0, keepdims=True)
        ctm2_s[...] = ct * (-2.0)
        csq_s[...] = jnp.broadcast_to(csq, (8, _K))

    f = f_ref[...]
    f_sq = jnp.sum(f * f, axis=1, keepdims=True)

    for r in range(_BN // _BR):
        rs = slice(r * _BR, (r + 1) * _BR)
        fr = f[rs, :]
        fsq_r = f_sq[rs, :]
        rmin = jnp.full((_BR, _LANES), jnp.inf, jnp.float32)
        rarg = jnp.zeros((_BR, _LANES), jnp.int32)
        for kb in range(_NKS):
            ksl = slice(kb * _BK, (kb + 1) * _BK)
            t = jnp.dot(fr, ctm2_s[:, ksl],
                        preferred_element_type=jnp.float32)
            csq_k = csq_s[0:1, ksl]
            for j in range(_BK // _LANES):
                sl = slice(j * _LANES, (j + 1) * _LANES)
                d2 = (fsq_r + csq_k[:, sl]) + t[:, sl]
                chunk = kb * (_BK // _LANES) + j
                lt = d2 < rmin
                rmin = jnp.minimum(d2, rmin)
                rarg = jnp.where(lt, chunk, rarg)
        gm = jnp.min(rmin, axis=1, keepdims=True)
        lane = lax.broadcasted_iota(jnp.int32, (_BR, _LANES), 1)
        fullidx = rarg * _LANES + lane
        cand = jnp.where(rmin == gm, fullidx, jnp.int32(2**30))
        out_ref[rs, :] = jnp.min(cand, axis=1, keepdims=True)


def _nearest_indices(features, ct):
    n, d = features.shape
    grid = (n // _BN,)
    return pl.pallas_call(
        _argmin_body,
        grid=grid,
        in_specs=[
            pl.BlockSpec((_BN, d), lambda i: (i, 0)),
            pl.BlockSpec((d, _K), lambda i: (0, 0)),
        ],
        out_specs=pl.BlockSpec((_BN, 1), lambda i: (i, 0)),
        out_shape=jax.ShapeDtypeStruct((n, 1), jnp.int32),
        scratch_shapes=[
            pltpu.VMEM((d, _K), jnp.float32),
            pltpu.VMEM((8, _K), jnp.float32),
        ],
        compiler_params=pltpu.CompilerParams(
            dimension_semantics=("arbitrary",),
        ),
    )(features, ct)


_NC = 2
_NS = 16
_NW = _NC * _NS
_CHUNK = 128


def _make_sc_gather(n, d):
    rows_per_w = n // (_NW * _CHUNK)
    b_per_w = rows_per_w * _CHUNK
    mesh = plsc.VectorSubcoreMesh(core_axis_name="c", subcore_axis_name="s")

    @functools.partial(
        pl.kernel,
        mesh=mesh,
        out_type=jax.ShapeDtypeStruct((n, d), jnp.float32),
        scratch_types=[
            pltpu.VMEM((rows_per_w, _CHUNK), jnp.int32),
            pltpu.VMEM((b_per_w, d), jnp.float32),
            pltpu.SemaphoreType.DMA,
        ],
        compiler_params=pltpu.CompilerParams(use_tc_tiling_on_sc=False),
    )
    def gather(codes_hbm, idx_hbm, out_hbm, idx_v, rows_v, sem):
        wid = lax.axis_index("s") * _NC + lax.axis_index("c")
        pltpu.sync_copy(idx_hbm.at[wid], idx_v)
        copies = [
            pltpu.async_copy(
                codes_hbm.at[idx_v.at[j]],
                rows_v.at[pl.ds(j * _CHUNK, _CHUNK)],
                sem,
            )
            for j in range(rows_per_w)
        ]
        for c in copies:
            c.wait()
        pltpu.sync_copy(rows_v, out_hbm.at[pl.ds(wid * b_per_w, b_per_w)])

    return gather


def kernel(features, codes):
    n, d = features.shape
    ct = codes.T
    idx = _nearest_indices(features, ct)
    idx2 = idx.reshape(_NW, n // (_NW * _CHUNK), _CHUNK)
    return _make_sc_gather(n, d)(codes, idx2)


# speedup vs baseline: 1.1583x; 1.0262x over previous
import functools

import jax
import jax.numpy as jnp
from jax import lax
from jax.experimental import pallas as pl
from jax.experimental.pallas import tpu as pltpu
from jax.experimental.pallas import tpu_sc as plsc


_BN = 1024
_BK = 512
_BR = 256
_LANES = 128


_K = 8192
_NKS = _K // _BK


def _argmin_body(f_ref, ct_ref, out_ref, ctm2_s, csq_s):
    n = pl.program_id(0)

    @pl.when(n == 0)
    def _():
        ct = ct_ref[...]
        csq = jnp.sum(ct * ct, axis=0, keepdims=True)
        ctm2_s[...] = ct * (-2.0)
        csq_s[...] = jnp.broadcast_to(csq, (8, _K))

    f = f_ref[...]
    f_sq = jnp.sum(f * f, axis=1, keepdims=True)

    for r in range(_BN // _BR):
        rs = slice(r * _BR, (r + 1) * _BR)
        fr = f[rs, :]
        fsq_r = f_sq[rs, :]
        rmin = jnp.full((_BR, _LANES), jnp.inf, jnp.float32)
        rarg = jnp.zeros((_BR, _LANES), jnp.int32)
        for kb in range(_NKS):
            ksl = slice(kb * _BK, (kb + 1) * _BK)
            t = jnp.dot(fr, ctm2_s[:, ksl],
                        preferred_element_type=jnp.float32)
            csq_k = csq_s[0:1, ksl]
            for j in range(_BK // _LANES):
                sl = slice(j * _LANES, (j + 1) * _LANES)
                d2 = (fsq_r + csq_k[:, sl]) + t[:, sl]
                chunk = kb * (_BK // _LANES) + j
                lt = d2 < rmin
                rmin = jnp.minimum(d2, rmin)
                rarg = jnp.where(lt, chunk, rarg)
        gm = jnp.min(rmin, axis=1, keepdims=True)
        lane = lax.broadcasted_iota(jnp.int32, (_BR, _LANES), 1)
        fullidx = rarg * _LANES + lane
        cand = jnp.where(rmin == gm, fullidx, jnp.int32(2**30))
        out_ref[rs, :] = jnp.min(cand, axis=1, keepdims=True)


def _nearest_indices(features, ct):
    n, d = features.shape
    grid = (n // _BN,)
    return pl.pallas_call(
        _argmin_body,
        grid=grid,
        in_specs=[
            pl.BlockSpec((_BN, d), lambda i: (i, 0)),
            pl.BlockSpec((d, _K), lambda i: (0, 0)),
        ],
        out_specs=pl.BlockSpec((_BN, 1), lambda i: (i, 0)),
        out_shape=jax.ShapeDtypeStruct((n, 1), jnp.int32),
        scratch_shapes=[
            pltpu.VMEM((d, _K), jnp.float32),
            pltpu.VMEM((8, _K), jnp.float32),
        ],
        compiler_params=pltpu.CompilerParams(
            dimension_semantics=("arbitrary",),
        ),
    )(features, ct)


_NC = 2
_NS = 16
_NW = _NC * _NS
_CHUNK = 128


def _make_sc_gather(n, d):
    rows_per_w = n // (_NW * _CHUNK)
    b_per_w = rows_per_w * _CHUNK
    mesh = plsc.VectorSubcoreMesh(core_axis_name="c", subcore_axis_name="s")

    @functools.partial(
        pl.kernel,
        mesh=mesh,
        out_type=jax.ShapeDtypeStruct((n, d), jnp.float32),
        scratch_types=[
            pltpu.VMEM((rows_per_w, _CHUNK), jnp.int32),
            pltpu.VMEM((b_per_w, d), jnp.float32),
            pltpu.SemaphoreType.DMA,
        ],
        compiler_params=pltpu.CompilerParams(use_tc_tiling_on_sc=False),
    )
    def gather(codes_hbm, idx_hbm, out_hbm, idx_v, rows_v, sem):
        wid = lax.axis_index("s") * _NC + lax.axis_index("c")
        pltpu.sync_copy(idx_hbm.at[wid], idx_v)
        copies = [
            pltpu.async_copy(
                codes_hbm.at[idx_v.at[j]],
                rows_v.at[pl.ds(j * _CHUNK, _CHUNK)],
                sem,
            )
            for j in range(rows_per_w)
        ]
        for c in copies:
            c.wait()
        pltpu.sync_copy(rows_v, out_hbm.at[pl.ds(wid * b_per_w, b_per_w)])

    return gather


def kernel(features, codes):
    n, d = features.shape
    ct = codes.T
    idx = _nearest_indices(features, ct)
    idx2 = idx.reshape(_NW, n // (_NW * _CHUNK), _CHUNK)
    return _make_sc_gather(n, d)(codes, idx2)


# speedup vs baseline: 1.1844x; 1.0225x over previous
import functools

import jax
import jax.numpy as jnp
from jax import lax
from jax.experimental import pallas as pl
from jax.experimental.pallas import tpu as pltpu
from jax.experimental.pallas import tpu_sc as plsc


_BN = 1024
_BK = 512
_BR = 512
_LANES = 128


_K = 8192
_NKS = _K // _BK


def _argmin_body(f_ref, ct_ref, out_ref, ctm2_s, csq_s):
    n = pl.program_id(0)

    @pl.when(n == 0)
    def _():
        ct = ct_ref[...]
        csq = jnp.sum(ct * ct, axis=0, keepdims=True)
        ctm2_s[...] = ct * (-2.0)
        csq_s[...] = jnp.broadcast_to(csq, (8, _K))

    f = f_ref[...]
    f_sq = jnp.sum(f * f, axis=1, keepdims=True)

    for r in range(_BN // _BR):
        rs = slice(r * _BR, (r + 1) * _BR)
        fr = f[rs, :]
        fsq_r = f_sq[rs, :]
        rmin = jnp.full((_BR, _LANES), jnp.inf, jnp.float32)
        rarg = jnp.zeros((_BR, _LANES), jnp.int32)
        for kb in range(_NKS):
            ksl = slice(kb * _BK, (kb + 1) * _BK)
            t = jnp.dot(fr, ctm2_s[:, ksl],
                        preferred_element_type=jnp.float32)
            csq_k = csq_s[0:1, ksl]
            for j in range(_BK // _LANES):
                sl = slice(j * _LANES, (j + 1) * _LANES)
                d2 = (fsq_r + csq_k[:, sl]) + t[:, sl]
                chunk = kb * (_BK // _LANES) + j
                lt = d2 < rmin
                rmin = jnp.minimum(d2, rmin)
                rarg = jnp.where(lt, chunk, rarg)
        gm = jnp.min(rmin, axis=1, keepdims=True)
        lane = lax.broadcasted_iota(jnp.int32, (_BR, _LANES), 1)
        fullidx = rarg * _LANES + lane
        cand = jnp.where(rmin == gm, fullidx, jnp.int32(2**30))
        out_ref[rs, :] = jnp.min(cand, axis=1, keepdims=True)


def _nearest_indices(features, ct):
    n, d = features.shape
    grid = (n // _BN,)
    return pl.pallas_call(
        _argmin_body,
        grid=grid,
        in_specs=[
            pl.BlockSpec((_BN, d), lambda i: (i, 0)),
            pl.BlockSpec((d, _K), lambda i: (0, 0)),
        ],
        out_specs=pl.BlockSpec((_BN, 1), lambda i: (i, 0)),
        out_shape=jax.ShapeDtypeStruct((n, 1), jnp.int32),
        scratch_shapes=[
            pltpu.VMEM((d, _K), jnp.float32),
            pltpu.VMEM((8, _K), jnp.float32),
        ],
        compiler_params=pltpu.CompilerParams(
            dimension_semantics=("arbitrary",),
        ),
    )(features, ct)


_NC = 2
_NS = 16
_NW = _NC * _NS
_CHUNK = 128


def _make_sc_gather(n, d):
    rows_per_w = n // (_NW * _CHUNK)
    b_per_w = rows_per_w * _CHUNK
    mesh = plsc.VectorSubcoreMesh(core_axis_name="c", subcore_axis_name="s")

    @functools.partial(
        pl.kernel,
        mesh=mesh,
        out_type=jax.ShapeDtypeStruct((n, d), jnp.float32),
        scratch_types=[
            pltpu.VMEM((rows_per_w, _CHUNK), jnp.int32),
            pltpu.VMEM((b_per_w, d), jnp.float32),
            pltpu.SemaphoreType.DMA,
        ],
        compiler_params=pltpu.CompilerParams(use_tc_tiling_on_sc=False),
    )
    def gather(codes_hbm, idx_hbm, out_hbm, idx_v, rows_v, sem):
        wid = lax.axis_index("s") * _NC + lax.axis_index("c")
        pltpu.sync_copy(idx_hbm.at[wid], idx_v)
        copies = [
            pltpu.async_copy(
                codes_hbm.at[idx_v.at[j]],
                rows_v.at[pl.ds(j * _CHUNK, _CHUNK)],
                sem,
            )
            for j in range(rows_per_w)
        ]
        for c in copies:
            c.wait()
        pltpu.sync_copy(rows_v, out_hbm.at[pl.ds(wid * b_per_w, b_per_w)])

    return gather


def kernel(features, codes):
    n, d = features.shape
    ct = codes.T
    idx = _nearest_indices(features, ct)
    idx2 = idx.reshape(_NW, n // (_NW * _CHUNK), _CHUNK)
    return _make_sc_gather(n, d)(codes, idx2)


# speedup vs baseline: 1.1910x; 1.0056x over previous
import functools

import jax
import jax.numpy as jnp
from jax import lax
from jax.experimental import pallas as pl
from jax.experimental.pallas import tpu as pltpu
from jax.experimental.pallas import tpu_sc as plsc


_BN = 1536
_BK = 512
_BR = 512
_LANES = 128


_K = 8192
_NKS = _K // _BK


def _argmin_body(f_ref, ct_ref, out_ref, ctm2_s, csq_s):
    n = pl.program_id(0)

    @pl.when(n == 0)
    def _():
        ct = ct_ref[...]
        csq = jnp.sum(ct * ct, axis=0, keepdims=True)
        ctm2_s[...] = ct * (-2.0)
        csq_s[...] = jnp.broadcast_to(csq, (8, _K))

    f = f_ref[...]
    f_sq = jnp.sum(f * f, axis=1, keepdims=True)

    for r in range(_BN // _BR):
        rs = slice(r * _BR, (r + 1) * _BR)
        fr = f[rs, :]
        fsq_r = f_sq[rs, :]
        rmin = jnp.full((_BR, _LANES), jnp.inf, jnp.float32)
        rarg = jnp.zeros((_BR, _LANES), jnp.int32)
        for kb in range(_NKS):
            ksl = slice(kb * _BK, (kb + 1) * _BK)
            t = jnp.dot(fr, ctm2_s[:, ksl],
                        preferred_element_type=jnp.float32)
            csq_k = csq_s[0:1, ksl]
            for j in range(_BK // _LANES):
                sl = slice(j * _LANES, (j + 1) * _LANES)
                d2 = (fsq_r + csq_k[:, sl]) + t[:, sl]
                chunk = kb * (_BK // _LANES) + j
                lt = d2 < rmin
                rmin = jnp.minimum(d2, rmin)
                rarg = jnp.where(lt, chunk, rarg)
        gm = jnp.min(rmin, axis=1, keepdims=True)
        lane = lax.broadcasted_iota(jnp.int32, (_BR, _LANES), 1)
        fullidx = rarg * _LANES + lane
        cand = jnp.where(rmin == gm, fullidx, jnp.int32(2**30))
        out_ref[rs, :] = jnp.min(cand, axis=1, keepdims=True)


def _nearest_indices(features, ct):
    n, d = features.shape
    grid = (n // _BN,)
    return pl.pallas_call(
        _argmin_body,
        grid=grid,
        in_specs=[
            pl.BlockSpec((_BN, d), lambda i: (i, 0)),
            pl.BlockSpec((d, _K), lambda i: (0, 0)),
        ],
        out_specs=pl.BlockSpec((_BN, 1), lambda i: (i, 0)),
        out_shape=jax.ShapeDtypeStruct((n, 1), jnp.int32),
        scratch_shapes=[
            pltpu.VMEM((d, _K), jnp.float32),
            pltpu.VMEM((8, _K), jnp.float32),
        ],
        compiler_params=pltpu.CompilerParams(
            dimension_semantics=("arbitrary",),
        ),
    )(features, ct)


_NC = 2
_NS = 16
_NW = _NC * _NS
_CHUNK = 128


def _make_sc_gather(n, d):
    rows_per_w = n // (_NW * _CHUNK)
    b_per_w = rows_per_w * _CHUNK
    mesh = plsc.VectorSubcoreMesh(core_axis_name="c", subcore_axis_name="s")

    @functools.partial(
        pl.kernel,
        mesh=mesh,
        out_type=jax.ShapeDtypeStruct((n, d), jnp.float32),
        scratch_types=[
            pltpu.VMEM((rows_per_w, _CHUNK), jnp.int32),
            pltpu.VMEM((b_per_w, d), jnp.float32),
            pltpu.SemaphoreType.DMA,
        ],
        compiler_params=pltpu.CompilerParams(use_tc_tiling_on_sc=False),
    )
    def gather(codes_hbm, idx_hbm, out_hbm, idx_v, rows_v, sem):
        wid = lax.axis_index("s") * _NC + lax.axis_index("c")
        pltpu.sync_copy(idx_hbm.at[wid], idx_v)
        copies = [
            pltpu.async_copy(
                codes_hbm.at[idx_v.at[j]],
                rows_v.at[pl.ds(j * _CHUNK, _CHUNK)],
                sem,
            )
            for j in range(rows_per_w)
        ]
        for c in copies:
            c.wait()
        pltpu.sync_copy(rows_v, out_hbm.at[pl.ds(wid * b_per_w, b_per_w)])

    return gather


def kernel(features, codes):
    n, d = features.shape
    ct = codes.T
    idx = _nearest_indices(features, ct)
    idx2 = idx.reshape(_NW, n // (_NW * _CHUNK), _CHUNK)
    return _make_sc_gather(n, d)(codes, idx2)
